# Initial kernel scaffold; baseline (speedup 1.0000x reference)
import functools

import jax
import jax.numpy as jnp
from jax.experimental import pallas as pl

N_EMB = 1024
D = 64
B = 32
L = 576
_LOSS_SCALE = 1.25 / (B * L * D)


def _vq_body(z_ref, emb_ref, zq_ref, loss_ref, codes_ref):
    b = pl.program_id(0)
    z = z_ref[0]
    emb = emb_ref[...]

    z_sq = jnp.sum(z * z, axis=0, keepdims=True)
    e_sq = jnp.sum(emb * emb, axis=1, keepdims=True)
    m = jax.lax.dot_general(
        emb, z, (((1,), (0,)), ((), ())),
        preferred_element_type=jnp.float32)
    dist = (z_sq - 2.0 * m) + e_sq

    min_d = jnp.min(dist, axis=0, keepdims=True)
    k_iota = jax.lax.broadcasted_iota(jnp.int32, dist.shape, 0)
    codes = jnp.min(jnp.where(dist == min_d, k_iota, N_EMB), axis=0)
    codes_ref[0, 0, :] = codes

    onehot = (k_iota == codes[None, :]).astype(jnp.float32)
    zq = jax.lax.dot_general(
        emb, onehot, (((0,), (0,)), ((), ())),
        preferred_element_type=jnp.float32)
    zq_ref[0] = zq

    part = jnp.sum((zq - z) ** 2)

    @pl.when(b == 0)
    def _():
        loss_ref[0, 0] = 0.0

    total = loss_ref[0, 0] + part
    loss_ref[0, 0] = jnp.where(b == B - 1, total * _LOSS_SCALE, total)


@functools.partial(jax.jit, static_argnames=("interpret",))
def kernel(z, embedding, interpret=False):
    zq, loss, codes3 = pl.pallas_call(
        _vq_body,
        grid=(B,),
        in_specs=[
            pl.BlockSpec((1, D, L), lambda b: (b, 0, 0)),
            pl.BlockSpec((N_EMB, D), lambda b: (0, 0)),
        ],
        out_specs=[
            pl.BlockSpec((1, D, L), lambda b: (b, 0, 0)),
            pl.BlockSpec((1, 1), lambda b: (0, 0)),
            pl.BlockSpec((1, 1, L), lambda b: (b, 0, 0)),
        ],
        out_shape=[
            jax.ShapeDtypeStruct((B, D, L), jnp.float32),
            jax.ShapeDtypeStruct((1, 1), jnp.float32),
            jax.ShapeDtypeStruct((B, 1, L), jnp.int32),
        ],
        interpret=interpret,
    )(z, embedding)
    return zq, loss[0, 0], codes3.reshape(B, L)


# speedup vs baseline: 2.3567x; 2.3567x over previous
import functools

import jax
import jax.numpy as jnp
from jax.experimental import pallas as pl

N_EMB = 1024
D = 64
B = 32
L = 576
_LOSS_SCALE = 1.25 / (B * L * D)


def _vq_body(z_ref, emb_ref, zq_ref, loss_ref, codes_ref):
    b = pl.program_id(0)
    z = z_ref[0]
    emb = emb_ref[...]

    z_sq = jnp.sum(z * z, axis=0, keepdims=True)
    e_sq = jnp.sum(emb * emb, axis=1, keepdims=True)
    m = jax.lax.dot_general(
        emb, z, (((1,), (0,)), ((), ())),
        preferred_element_type=jnp.float32)
    dist = (z_sq - 2.0 * m) + e_sq

    min_d = jnp.min(dist, axis=0, keepdims=True)
    k_iota = jax.lax.broadcasted_iota(jnp.int32, dist.shape, 0)
    codes = jnp.min(jnp.where(dist == min_d, k_iota, N_EMB),
                    axis=0, keepdims=True)
    codes_ref[0] = codes

    onehot = (k_iota == codes).astype(jnp.float32)
    zq = jax.lax.dot_general(
        emb, onehot, (((0,), (0,)), ((), ())),
        preferred_element_type=jnp.float32)
    zq_ref[0] = zq

    sq = (zq - z) ** 2
    part = jnp.sum(jnp.sum(sq, axis=0, keepdims=True), axis=1,
                   keepdims=True)

    @pl.when(b == 0)
    def _():
        loss_ref[...] = jnp.zeros((1, 1), jnp.float32)

    total = loss_ref[...] + part
    loss_ref[...] = jnp.where(b == B - 1, total * _LOSS_SCALE, total)


@functools.partial(jax.jit, static_argnames=("interpret",))
def kernel(z, embedding, interpret=False):
    zq, loss, codes3 = pl.pallas_call(
        _vq_body,
        grid=(B,),
        in_specs=[
            pl.BlockSpec((1, D, L), lambda b: (b, 0, 0)),
            pl.BlockSpec((N_EMB, D), lambda b: (0, 0)),
        ],
        out_specs=[
            pl.BlockSpec((1, D, L), lambda b: (b, 0, 0)),
            pl.BlockSpec((1, 1), lambda b: (0, 0)),
            pl.BlockSpec((1, 1, L), lambda b: (b, 0, 0)),
        ],
        out_shape=[
            jax.ShapeDtypeStruct((B, D, L), jnp.float32),
            jax.ShapeDtypeStruct((1, 1), jnp.float32),
            jax.ShapeDtypeStruct((B, 1, L), jnp.int32),
        ],
        interpret=interpret,
    )(z, embedding)
    return zq, loss[0, 0], codes3.reshape(B, L)
